# baseline (speedup 1.0000x reference)
import functools

import jax
import jax.numpy as jnp
from jax import lax
from jax.experimental import pallas as pl
from jax.experimental.pallas import tpu as pltpu
from jax.experimental.pallas import tpu_sc as plsc

N = 10000
K = 16
E = N * K
D = 256
DE = 16
OUT = 32
H4 = 4 * OUT

_CH = 128
_NW = 32
_EPAD = ((E + _CH * _NW - 1) // (_CH * _NW)) * (_CH * _NW)


def _sc_gather(table, idx_pad, n_cols):
    per_w = _EPAD // _NW
    n_ch = per_w // _CH
    nbuf = 2 if n_cols > 128 else 4
    n_outer = n_ch // nbuf
    mesh = plsc.VectorSubcoreMesh(core_axis_name="c", subcore_axis_name="s")

    @functools.partial(
        pl.kernel,
        mesh=mesh,
        out_type=jax.ShapeDtypeStruct((_EPAD, n_cols), jnp.float32),
        scratch_types=[
            pltpu.VMEM((per_w,), jnp.int32),
            pltpu.VMEM((nbuf, _CH, n_cols), jnp.float32),
            [pltpu.SemaphoreType.DMA] * nbuf,
        ],
    )
    def gk(table_hbm, idx_hbm, out_hbm, idx_v, rows_v, sems):
        wid = lax.axis_index("s") * 2 + lax.axis_index("c")
        wbase = wid * per_w
        pltpu.sync_copy(idx_hbm.at[pl.ds(wbase, per_w)], idx_v)

        def outer(g, carry):
            copies = []
            for b in range(nbuf):
                off = (g * nbuf + b) * _CH
                cp = pltpu.async_copy(
                    table_hbm.at[idx_v.at[pl.ds(off, _CH)]],
                    rows_v.at[b], sems[b])
                copies.append((cp, off))
            for b in range(nbuf):
                cp, off = copies[b]
                cp.wait()
                pltpu.sync_copy(rows_v.at[b],
                                out_hbm.at[pl.ds(wbase + off, _CH)])
            return carry

        lax.fori_loop(0, n_outer, outer, 0)

    return gk(table, idx_pad)


def _conv1_body(xg_ref, x_ref, wih_ref, whh_ref, b_ref, sw_ref, sb_ref,
                nw_ref, w2i_ref, out_ref, hp_ref):
    b_nodes = x_ref.shape[0]
    xg = xg_ref[...].reshape(b_nodes, K, D)
    wih = wih_ref[...]
    whh = whh_ref[...]
    bias = b_ref[...]
    h = jnp.zeros((b_nodes, D), jnp.float32)
    c = jnp.zeros((b_nodes, D), jnp.float32)
    for t in range(K):
        xt = xg[:, t, :]
        g = (jnp.dot(xt, wih, preferred_element_type=jnp.float32)
             + jnp.dot(h, whh, preferred_element_type=jnp.float32) + bias)
        gi = jax.nn.sigmoid(g[:, 0:D])
        gf = jax.nn.sigmoid(g[:, D:2 * D])
        gg = jnp.tanh(g[:, 2 * D:3 * D])
        go = jax.nn.sigmoid(g[:, 3 * D:4 * D])
        c = gf * c + gi * gg
        h = go * jnp.tanh(c)
    out = (jnp.dot(x_ref[...], sw_ref[...], preferred_element_type=jnp.float32)
           + sb_ref[...]
           + jnp.dot(h, nw_ref[...], preferred_element_type=jnp.float32))
    out = jnp.maximum(out, 0.0)
    out_ref[...] = out
    hp_ref[...] = jnp.dot(out, w2i_ref[...], preferred_element_type=jnp.float32)


def _conv_small_body(xp_ref, x_ref, whh_ref, b_ref, sw_ref, sb_ref,
                     nw_ref, out_ref, *, mode):
    b_nodes = x_ref.shape[0]
    xp = xp_ref[...].reshape(b_nodes, K, H4)
    whh = whh_ref[...]
    bias = b_ref[...]
    h = jnp.zeros((b_nodes, OUT), jnp.float32)
    c = jnp.zeros((b_nodes, OUT), jnp.float32)
    for t in range(K):
        g = (xp[:, t, :]
             + jnp.dot(h, whh, preferred_element_type=jnp.float32) + bias)
        gi = jax.nn.sigmoid(g[:, 0:OUT])
        gf = jax.nn.sigmoid(g[:, OUT:2 * OUT])
        gg = jnp.tanh(g[:, 2 * OUT:3 * OUT])
        go = jax.nn.sigmoid(g[:, 3 * OUT:4 * OUT])
        c = gf * c + gi * gg
        h = go * jnp.tanh(c)
    out = (jnp.dot(x_ref[...], sw_ref[...], preferred_element_type=jnp.float32)
           + sb_ref[...]
           + jnp.dot(h, nw_ref[...], preferred_element_type=jnp.float32))
    if mode == "mid":
        out = jnp.maximum(out, 0.0)
        out_ref[...] = jnp.concatenate(
            [out, jnp.zeros((b_nodes, H4 - OUT), jnp.float32)], axis=1)
    else:
        out_ref[...] = out


def _nnconv_body(hg_ref, e_ref, wz_ref, ebm_ref, nb_ref, w2i_ref,
                 out_ref, hp_ref):
    b_nodes = out_ref.shape[0]
    hg = hg_ref[...][:, 0:OUT].reshape(b_nodes, K, OUT)
    ee = e_ref[...].reshape(b_nodes, K, DE)
    zs = jnp.zeros((b_nodes, OUT * DE), jnp.float32)
    for t in range(K):
        zt = hg[:, t, :, None] * ee[:, t, None, :]
        zs = zs + zt.reshape(b_nodes, OUT * DE)
    hs = jnp.sum(hg, axis=1)
    out = (jnp.dot(zs, wz_ref[...], preferred_element_type=jnp.float32)
           + jnp.dot(hs, ebm_ref[...], preferred_element_type=jnp.float32)
           + nb_ref[...])
    out = jnp.maximum(out, 0.0)
    out_ref[...] = out
    hp_ref[...] = jnp.dot(out, w2i_ref[...], preferred_element_type=jnp.float32)


def _pool_body(h_ref, g1w_ref, g1b_ref, g2wt_ref, g2b_ref, f1w_ref, f1b_ref,
               f2w_ref, f2b_ref, f3wt_ref, f3b_ref, out_ref):
    h = h_ref[...]
    gate1 = jnp.maximum(
        jnp.dot(h, g1w_ref[...], preferred_element_type=jnp.float32)
        + g1b_ref[...], 0.0)
    gate = (jnp.sum(gate1 * g2wt_ref[...], axis=-1, keepdims=True)
            + g2b_ref[...])
    m = jnp.max(gate)
    a = jnp.exp(gate - m)
    s = jnp.sum(a)
    r = jnp.sum(a * h, axis=0, keepdims=True) / s
    r = jnp.where(r > 0.0, r, jnp.exp(jnp.minimum(r, 0.0)) - 1.0)
    r = jnp.maximum(
        jnp.dot(r, f1w_ref[...], preferred_element_type=jnp.float32)
        + f1b_ref[...], 0.0)
    r = jnp.maximum(
        jnp.dot(r, f2w_ref[...], preferred_element_type=jnp.float32)
        + f2b_ref[...], 0.0)
    out_ref[...] = (jnp.sum(r * f3wt_ref[...], axis=-1, keepdims=True)
                    + f3b_ref[...])


_B1 = 200
_B2 = 400
_B3 = 200


def _conv1(xg, x, w1i, w1h, b1, s1w, s1b, n1w, w2i):
    grid = N // _B1
    return pl.pallas_call(
        _conv1_body,
        grid=(grid,),
        in_specs=[
            pl.BlockSpec((_B1 * K, D), lambda i: (i, 0)),
            pl.BlockSpec((_B1, D), lambda i: (i, 0)),
            pl.BlockSpec((D, 4 * D), lambda i: (0, 0)),
            pl.BlockSpec((D, 4 * D), lambda i: (0, 0)),
            pl.BlockSpec((1, 4 * D), lambda i: (0, 0)),
            pl.BlockSpec((D, OUT), lambda i: (0, 0)),
            pl.BlockSpec((1, OUT), lambda i: (0, 0)),
            pl.BlockSpec((D, OUT), lambda i: (0, 0)),
            pl.BlockSpec((OUT, H4), lambda i: (0, 0)),
        ],
        out_specs=[
            pl.BlockSpec((_B1, OUT), lambda i: (i, 0)),
            pl.BlockSpec((_B1, H4), lambda i: (i, 0)),
        ],
        out_shape=[
            jax.ShapeDtypeStruct((N, OUT), jnp.float32),
            jax.ShapeDtypeStruct((N, H4), jnp.float32),
        ],
    )(xg, x, w1i, w1h, b1, s1w, s1b, n1w, w2i)


def _conv_small(xp, x, w2h, b2, s2w, s2b, n2w, mode):
    grid = N // _B2
    width = H4 if mode == "mid" else OUT
    return pl.pallas_call(
        functools.partial(_conv_small_body, mode=mode),
        grid=(grid,),
        in_specs=[
            pl.BlockSpec((_B2 * K, H4), lambda i: (i, 0)),
            pl.BlockSpec((_B2, OUT), lambda i: (i, 0)),
            pl.BlockSpec((OUT, H4), lambda i: (0, 0)),
            pl.BlockSpec((1, H4), lambda i: (0, 0)),
            pl.BlockSpec((OUT, OUT), lambda i: (0, 0)),
            pl.BlockSpec((1, OUT), lambda i: (0, 0)),
            pl.BlockSpec((OUT, OUT), lambda i: (0, 0)),
        ],
        out_specs=pl.BlockSpec((_B2, width), lambda i: (i, 0)),
        out_shape=jax.ShapeDtypeStruct((N, width), jnp.float32),
    )(xp, x, w2h, b2, s2w, s2b, n2w)


def _nnconv(hg, e, wz, ebm, nbb, w2i):
    grid = N // _B3
    return pl.pallas_call(
        _nnconv_body,
        grid=(grid,),
        in_specs=[
            pl.BlockSpec((_B3 * K, H4), lambda i: (i, 0)),
            pl.BlockSpec((_B3 * K, DE), lambda i: (i, 0)),
            pl.BlockSpec((OUT * DE, OUT), lambda i: (0, 0)),
            pl.BlockSpec((OUT, OUT), lambda i: (0, 0)),
            pl.BlockSpec((1, OUT), lambda i: (0, 0)),
            pl.BlockSpec((OUT, H4), lambda i: (0, 0)),
        ],
        out_specs=[
            pl.BlockSpec((_B3, OUT), lambda i: (i, 0)),
            pl.BlockSpec((_B3, H4), lambda i: (i, 0)),
        ],
        out_shape=[
            jax.ShapeDtypeStruct((N, OUT), jnp.float32),
            jax.ShapeDtypeStruct((N, H4), jnp.float32),
        ],
    )(hg, e, wz, ebm, nbb, w2i)


def _pool(h4, g1w, g1b, g2wt, g2b, f1w, f1b, f2w, f2b, f3wt, f3b):
    return pl.pallas_call(
        _pool_body,
        out_shape=jax.ShapeDtypeStruct((1, 1), jnp.float32),
    )(h4, g1w, g1b, g2wt, g2b, f1w, f1b, f2w, f2b, f3wt, f3b)


def kernel(x, edge_index, e, l1Wih, l1Whh, l1bih, l1bhh, s1w, s1b, n1w,
           l2Wih, l2Whh, l2bih, l2bhh, s2w, s2b, n2w, ew, eb, nb,
           g1w, g1b, g2w, g2b, f1w, f1b, f2w, f2b, f3w, f3b):
    src = edge_index[0].astype(jnp.int32)
    idx_pad = jnp.concatenate(
        [src, jnp.zeros((_EPAD - E,), jnp.int32)])

    w1i = l1Wih.T
    w1h = l1Whh.T
    b1 = (l1bih + l1bhh).reshape(1, 4 * D)
    w2i = l2Wih.T
    w2h = l2Whh.T
    b2 = (l2bih + l2bhh).reshape(1, H4)
    s1b2 = s1b.reshape(1, OUT)
    s2b2 = s2b.reshape(1, OUT)
    wz = ew.reshape(DE, OUT, OUT).transpose(1, 0, 2).reshape(OUT * DE, OUT)
    ebm = eb.reshape(OUT, OUT)
    nbb = nb.reshape(1, OUT)
    g1b2 = g1b.reshape(1, 16)
    g2wt = g2w.reshape(1, 16)
    g2b2 = g2b.reshape(1, 1)
    f1b2 = f1b.reshape(1, 32)
    f2b2 = f2b.reshape(1, 32)
    f3wt = f3w.reshape(1, 32)
    f3b2 = f3b.reshape(1, 1)

    xg = _sc_gather(x, idx_pad, D)
    h1, hp1 = _conv1(xg, x, w1i, w1h, b1, s1w, s1b2, n1w, w2i)
    xp2 = _sc_gather(hp1, idx_pad, H4)
    h2w = _conv_small(xp2, h1, w2h, b2, s2w, s2b2, n2w, mode="mid")
    hg2 = _sc_gather(h2w, idx_pad, H4)
    h3, hp3 = _nnconv(hg2, e, wz, ebm, nbb, w2i)
    xp4 = _sc_gather(hp3, idx_pad, H4)
    h4 = _conv_small(xp4, h3, w2h, b2, s2w, s2b2, n2w, mode="last")
    return _pool(h4, g1w, g1b2, g2wt, g2b2, f1w, f1b2, f2w, f2b2, f3wt, f3b2)


# speedup vs baseline: 1.0695x; 1.0695x over previous
import functools

import jax
import jax.numpy as jnp
from jax import lax
from jax.experimental import pallas as pl
from jax.experimental.pallas import tpu as pltpu
from jax.experimental.pallas import tpu_sc as plsc

N = 10000
K = 16
E = N * K
D = 256
DE = 16
OUT = 32
H4 = 4 * OUT

_CH = 128
_NW = 32
_EPAD = ((E + _CH * _NW - 1) // (_CH * _NW)) * (_CH * _NW)


def _sc_gather(table, idx_pad, n_cols):
    n_ch = _EPAD // (_CH * _NW)
    nbuf = 2 if n_cols > 128 else 4
    n_outer = n_ch // nbuf
    mesh = plsc.VectorSubcoreMesh(core_axis_name="c", subcore_axis_name="s")

    @functools.partial(
        pl.kernel,
        mesh=mesh,
        out_type=jax.ShapeDtypeStruct((_EPAD, n_cols), jnp.float32),
        scratch_types=[
            pltpu.VMEM((nbuf, _CH), jnp.int32),
            pltpu.VMEM((nbuf, _CH, n_cols), jnp.float32),
            [pltpu.SemaphoreType.DMA] * nbuf,
        ],
    )
    def gk(table_hbm, idx_hbm, out_hbm, idx_v, rows_v, sems):
        wid = lax.axis_index("s") * 2 + lax.axis_index("c")

        def outer(g, carry):
            copies = []
            for b in range(nbuf):
                base = ((g * nbuf + b) * _NW + wid) * _CH
                pltpu.sync_copy(idx_hbm.at[pl.ds(base, _CH)], idx_v.at[b])
                cp = pltpu.async_copy(
                    table_hbm.at[idx_v.at[b]], rows_v.at[b], sems[b])
                copies.append((cp, base))
            for b in range(nbuf):
                cp, base = copies[b]
                cp.wait()
                pltpu.sync_copy(rows_v.at[b], out_hbm.at[pl.ds(base, _CH)])
            return carry

        lax.fori_loop(0, n_outer, outer, 0)

    return gk(table, idx_pad)


def _conv1_body(xg_ref, x_ref, wih_ref, whh_ref, b_ref, sw_ref, sb_ref,
                nw_ref, w2i_ref, out_ref, hp_ref):
    b_nodes = x_ref.shape[0]
    xg = xg_ref[...].reshape(b_nodes, K, D)
    wih = wih_ref[...]
    whh = whh_ref[...]
    bias = b_ref[...]
    h = jnp.zeros((b_nodes, D), jnp.float32)
    c = jnp.zeros((b_nodes, D), jnp.float32)
    for t in range(K):
        xt = xg[:, t, :]
        g = (jnp.dot(xt, wih, preferred_element_type=jnp.float32)
             + jnp.dot(h, whh, preferred_element_type=jnp.float32) + bias)
        gi = jax.nn.sigmoid(g[:, 0:D])
        gf = jax.nn.sigmoid(g[:, D:2 * D])
        gg = jnp.tanh(g[:, 2 * D:3 * D])
        go = jax.nn.sigmoid(g[:, 3 * D:4 * D])
        c = gf * c + gi * gg
        h = go * jnp.tanh(c)
    out = (jnp.dot(x_ref[...], sw_ref[...], preferred_element_type=jnp.float32)
           + sb_ref[...]
           + jnp.dot(h, nw_ref[...], preferred_element_type=jnp.float32))
    out = jnp.maximum(out, 0.0)
    out_ref[...] = out
    hp_ref[...] = jnp.dot(out, w2i_ref[...], preferred_element_type=jnp.float32)


def _conv_small_body(xp_ref, x_ref, whh_ref, b_ref, sw_ref, sb_ref,
                     nw_ref, out_ref, *, mode):
    b_nodes = x_ref.shape[0]
    xp = xp_ref[...].reshape(b_nodes, K, H4)
    whh = whh_ref[...]
    bias = b_ref[...]
    h = jnp.zeros((b_nodes, OUT), jnp.float32)
    c = jnp.zeros((b_nodes, OUT), jnp.float32)
    for t in range(K):
        g = (xp[:, t, :]
             + jnp.dot(h, whh, preferred_element_type=jnp.float32) + bias)
        gi = jax.nn.sigmoid(g[:, 0:OUT])
        gf = jax.nn.sigmoid(g[:, OUT:2 * OUT])
        gg = jnp.tanh(g[:, 2 * OUT:3 * OUT])
        go = jax.nn.sigmoid(g[:, 3 * OUT:4 * OUT])
        c = gf * c + gi * gg
        h = go * jnp.tanh(c)
    out = (jnp.dot(x_ref[...], sw_ref[...], preferred_element_type=jnp.float32)
           + sb_ref[...]
           + jnp.dot(h, nw_ref[...], preferred_element_type=jnp.float32))
    if mode == "mid":
        out = jnp.maximum(out, 0.0)
        out_ref[...] = jnp.concatenate(
            [out, jnp.zeros((b_nodes, H4 - OUT), jnp.float32)], axis=1)
    else:
        out_ref[...] = out


def _nnconv_body(hg_ref, e_ref, wz_ref, ebm_ref, nb_ref, w2i_ref,
                 out_ref, hp_ref):
    b_nodes = out_ref.shape[0]
    hg = hg_ref[...][:, 0:OUT].reshape(b_nodes, K, OUT)
    ee = e_ref[...].reshape(b_nodes, K, DE)
    zs = jnp.zeros((b_nodes, OUT * DE), jnp.float32)
    for t in range(K):
        zt = hg[:, t, :, None] * ee[:, t, None, :]
        zs = zs + zt.reshape(b_nodes, OUT * DE)
    hs = jnp.sum(hg, axis=1)
    out = (jnp.dot(zs, wz_ref[...], preferred_element_type=jnp.float32)
           + jnp.dot(hs, ebm_ref[...], preferred_element_type=jnp.float32)
           + nb_ref[...])
    out = jnp.maximum(out, 0.0)
    out_ref[...] = out
    hp_ref[...] = jnp.dot(out, w2i_ref[...], preferred_element_type=jnp.float32)


def _pool_body(h_ref, g1w_ref, g1b_ref, g2wt_ref, g2b_ref, f1w_ref, f1b_ref,
               f2w_ref, f2b_ref, f3wt_ref, f3b_ref, out_ref):
    h = h_ref[...]
    gate1 = jnp.maximum(
        jnp.dot(h, g1w_ref[...], preferred_element_type=jnp.float32)
        + g1b_ref[...], 0.0)
    gate = (jnp.sum(gate1 * g2wt_ref[...], axis=-1, keepdims=True)
            + g2b_ref[...])
    m = jnp.max(gate)
    a = jnp.exp(gate - m)
    s = jnp.sum(a)
    r = jnp.sum(a * h, axis=0, keepdims=True) / s
    r = jnp.where(r > 0.0, r, jnp.exp(jnp.minimum(r, 0.0)) - 1.0)
    r = jnp.maximum(
        jnp.dot(r, f1w_ref[...], preferred_element_type=jnp.float32)
        + f1b_ref[...], 0.0)
    r = jnp.maximum(
        jnp.dot(r, f2w_ref[...], preferred_element_type=jnp.float32)
        + f2b_ref[...], 0.0)
    out_ref[...] = (jnp.sum(r * f3wt_ref[...], axis=-1, keepdims=True)
                    + f3b_ref[...])


_B1 = 200
_B2 = 400
_B3 = 200


def _conv1(xg, x, w1i, w1h, b1, s1w, s1b, n1w, w2i):
    grid = N // _B1
    return pl.pallas_call(
        _conv1_body,
        grid=(grid,),
        in_specs=[
            pl.BlockSpec((_B1 * K, D), lambda i: (i, 0)),
            pl.BlockSpec((_B1, D), lambda i: (i, 0)),
            pl.BlockSpec((D, 4 * D), lambda i: (0, 0)),
            pl.BlockSpec((D, 4 * D), lambda i: (0, 0)),
            pl.BlockSpec((1, 4 * D), lambda i: (0, 0)),
            pl.BlockSpec((D, OUT), lambda i: (0, 0)),
            pl.BlockSpec((1, OUT), lambda i: (0, 0)),
            pl.BlockSpec((D, OUT), lambda i: (0, 0)),
            pl.BlockSpec((OUT, H4), lambda i: (0, 0)),
        ],
        out_specs=[
            pl.BlockSpec((_B1, OUT), lambda i: (i, 0)),
            pl.BlockSpec((_B1, H4), lambda i: (i, 0)),
        ],
        out_shape=[
            jax.ShapeDtypeStruct((N, OUT), jnp.float32),
            jax.ShapeDtypeStruct((N, H4), jnp.float32),
        ],
    )(xg, x, w1i, w1h, b1, s1w, s1b, n1w, w2i)


def _conv_small(xp, x, w2h, b2, s2w, s2b, n2w, mode):
    grid = N // _B2
    width = H4 if mode == "mid" else OUT
    return pl.pallas_call(
        functools.partial(_conv_small_body, mode=mode),
        grid=(grid,),
        in_specs=[
            pl.BlockSpec((_B2 * K, H4), lambda i: (i, 0)),
            pl.BlockSpec((_B2, OUT), lambda i: (i, 0)),
            pl.BlockSpec((OUT, H4), lambda i: (0, 0)),
            pl.BlockSpec((1, H4), lambda i: (0, 0)),
            pl.BlockSpec((OUT, OUT), lambda i: (0, 0)),
            pl.BlockSpec((1, OUT), lambda i: (0, 0)),
            pl.BlockSpec((OUT, OUT), lambda i: (0, 0)),
        ],
        out_specs=pl.BlockSpec((_B2, width), lambda i: (i, 0)),
        out_shape=jax.ShapeDtypeStruct((N, width), jnp.float32),
    )(xp, x, w2h, b2, s2w, s2b, n2w)


def _nnconv(hg, e, wz, ebm, nbb, w2i):
    grid = N // _B3
    return pl.pallas_call(
        _nnconv_body,
        grid=(grid,),
        in_specs=[
            pl.BlockSpec((_B3 * K, H4), lambda i: (i, 0)),
            pl.BlockSpec((_B3 * K, DE), lambda i: (i, 0)),
            pl.BlockSpec((OUT * DE, OUT), lambda i: (0, 0)),
            pl.BlockSpec((OUT, OUT), lambda i: (0, 0)),
            pl.BlockSpec((1, OUT), lambda i: (0, 0)),
            pl.BlockSpec((OUT, H4), lambda i: (0, 0)),
        ],
        out_specs=[
            pl.BlockSpec((_B3, OUT), lambda i: (i, 0)),
            pl.BlockSpec((_B3, H4), lambda i: (i, 0)),
        ],
        out_shape=[
            jax.ShapeDtypeStruct((N, OUT), jnp.float32),
            jax.ShapeDtypeStruct((N, H4), jnp.float32),
        ],
    )(hg, e, wz, ebm, nbb, w2i)


def _pool(h4, g1w, g1b, g2wt, g2b, f1w, f1b, f2w, f2b, f3wt, f3b):
    return pl.pallas_call(
        _pool_body,
        out_shape=jax.ShapeDtypeStruct((1, 1), jnp.float32),
    )(h4, g1w, g1b, g2wt, g2b, f1w, f1b, f2w, f2b, f3wt, f3b)


def kernel(x, edge_index, e, l1Wih, l1Whh, l1bih, l1bhh, s1w, s1b, n1w,
           l2Wih, l2Whh, l2bih, l2bhh, s2w, s2b, n2w, ew, eb, nb,
           g1w, g1b, g2w, g2b, f1w, f1b, f2w, f2b, f3w, f3b):
    src = edge_index[0].astype(jnp.int32)
    idx_pad = jnp.concatenate(
        [src, jnp.zeros((_EPAD - E,), jnp.int32)])

    w1i = l1Wih.T
    w1h = l1Whh.T
    b1 = (l1bih + l1bhh).reshape(1, 4 * D)
    w2i = l2Wih.T
    w2h = l2Whh.T
    b2 = (l2bih + l2bhh).reshape(1, H4)
    s1b2 = s1b.reshape(1, OUT)
    s2b2 = s2b.reshape(1, OUT)
    wz = ew.reshape(DE, OUT, OUT).transpose(1, 0, 2).reshape(OUT * DE, OUT)
    ebm = eb.reshape(OUT, OUT)
    nbb = nb.reshape(1, OUT)
    g1b2 = g1b.reshape(1, 16)
    g2wt = g2w.reshape(1, 16)
    g2b2 = g2b.reshape(1, 1)
    f1b2 = f1b.reshape(1, 32)
    f2b2 = f2b.reshape(1, 32)
    f3wt = f3w.reshape(1, 32)
    f3b2 = f3b.reshape(1, 1)

    xg = _sc_gather(x, idx_pad, D)
    h1, hp1 = _conv1(xg, x, w1i, w1h, b1, s1w, s1b2, n1w, w2i)
    xp2 = _sc_gather(hp1, idx_pad, H4)
    h2w = _conv_small(xp2, h1, w2h, b2, s2w, s2b2, n2w, mode="mid")
    hg2 = _sc_gather(h2w, idx_pad, H4)
    h3, hp3 = _nnconv(hg2, e, wz, ebm, nbb, w2i)
    xp4 = _sc_gather(hp3, idx_pad, H4)
    h4 = _conv_small(xp4, h3, w2h, b2, s2w, s2b2, n2w, mode="last")
    return _pool(h4, g1w, g1b2, g2wt, g2b2, f1w, f1b2, f2w, f2b2, f3wt, f3b2)
